# Initial kernel scaffold; baseline (speedup 1.0000x reference)
import functools

import jax
import jax.numpy as jnp
from jax import lax
from jax.experimental import pallas as pl
from jax.experimental.pallas import tpu as pltpu
from jax.experimental.pallas import tpu_sc as plsc

N_NODES = 10000
N_EDGES = 320000
D_FEAT = 128
OUT_SIZE = 128

NC = 2
NS = 16
NW = NC * NS
EPW = N_EDGES // NW
K = 80
NCHUNK = EPW // K
RPT = N_NODES // NS


def _make_sc_agg(dt):
    mesh = plsc.VectorSubcoreMesh(core_axis_name="c", subcore_axis_name="s")

    @functools.partial(
        pl.kernel,
        mesh=mesh,
        out_type=jax.ShapeDtypeStruct((NC, N_NODES, dt), jnp.float32),
        scratch_types=[
            pltpu.VMEM((NCHUNK, 2, K), jnp.int32),
            pltpu.VMEM((K, dt), jnp.float32),
            pltpu.VMEM_SHARED((N_NODES, dt), jnp.float32),
            pltpu.SemaphoreType.DMA,
        ],
    )
    def sc_agg(tab_h, ei_h, z_h, out_p, idx_v, rows_v, acc_sh, sem):
        c = lax.axis_index("c")
        s = lax.axis_index("s")
        wid = c * NS + s
        rows = pl.ds(s * RPT, RPT)

        pltpu.sync_copy(z_h.at[rows], acc_sh.at[rows])
        pltpu.sync_copy(ei_h.at[wid], idx_v)
        plsc.subcore_barrier()

        def chunk(i, carry):
            src_idx = idx_v.at[i, 0]
            dst_idx = idx_v.at[i, 1]
            pltpu.async_copy(tab_h.at[src_idx], rows_v, sem).wait()
            pltpu.sync_copy(rows_v, acc_sh.at[dst_idx], add=True)
            return carry

        lax.fori_loop(0, NCHUNK, chunk, 0)

        plsc.subcore_barrier()
        pltpu.sync_copy(acc_sh.at[rows], out_p.at[c, rows])

    return sc_agg


_sc_agg_144 = _make_sc_agg(D_FEAT + 16)
_sc_agg_128 = _make_sc_agg(D_FEAT)

_TC_R = 2000


def _tc1_body(h_ref, p_ref, w_ref, o_ref):
    h = h_ref[...]
    p = p_ref[0] + p_ref[1]
    d = jnp.maximum(p[:, D_FEAT:D_FEAT + 1], 1e-12)
    agg = p[:, :D_FEAT] / d
    wh = w_ref[:, :D_FEAT]
    wa = w_ref[:, D_FEAT:]
    acc = lax.dot_general(h, wh, (((1,), (1,)), ((), ())),
                          preferred_element_type=jnp.float32)
    acc = acc + lax.dot_general(agg, wa, (((1,), (1,)), ((), ())),
                                preferred_element_type=jnp.float32)
    o_ref[...] = jnp.maximum(acc, 0.0)


def _tc2_body(h_ref, p_ref, dp_ref, w_ref, o_ref):
    h = h_ref[...]
    p = p_ref[0] + p_ref[1]
    d = jnp.maximum(dp_ref[0] + dp_ref[1], 1e-12)
    agg = p / d
    wh = w_ref[:, :D_FEAT]
    wa = w_ref[:, D_FEAT:]
    acc = lax.dot_general(h, wh, (((1,), (1,)), ((), ())),
                          preferred_element_type=jnp.float32)
    acc = acc + lax.dot_general(agg, wa, (((1,), (1,)), ((), ())),
                                preferred_element_type=jnp.float32)
    o_ref[...] = jnp.maximum(acc, 0.0)


def _tc_layer1(h, p, W):
    grid = (N_NODES // _TC_R,)
    return pl.pallas_call(
        _tc1_body,
        grid=grid,
        in_specs=[
            pl.BlockSpec((_TC_R, D_FEAT), lambda i: (i, 0)),
            pl.BlockSpec((NC, _TC_R, D_FEAT + 16), lambda i: (0, i, 0)),
            pl.BlockSpec((OUT_SIZE, 2 * D_FEAT), lambda i: (0, 0)),
        ],
        out_specs=pl.BlockSpec((_TC_R, OUT_SIZE), lambda i: (i, 0)),
        out_shape=jax.ShapeDtypeStruct((N_NODES, OUT_SIZE), jnp.float32),
    )(h, p, W)


def _tc_layer2(h, p, dp, W):
    grid = (N_NODES // _TC_R,)
    return pl.pallas_call(
        _tc2_body,
        grid=grid,
        in_specs=[
            pl.BlockSpec((_TC_R, D_FEAT), lambda i: (i, 0)),
            pl.BlockSpec((NC, _TC_R, D_FEAT), lambda i: (0, i, 0)),
            pl.BlockSpec((NC, _TC_R, 1), lambda i: (0, i, 0)),
            pl.BlockSpec((OUT_SIZE, 2 * OUT_SIZE), lambda i: (0, 0)),
        ],
        out_specs=pl.BlockSpec((_TC_R, OUT_SIZE), lambda i: (i, 0)),
        out_shape=jax.ShapeDtypeStruct((N_NODES, OUT_SIZE), jnp.float32),
    )(h, p, dp, W)


def kernel(x, W1, W2, edge_index):
    ei = jnp.transpose(
        edge_index.reshape(2, NW, NCHUNK, K), (1, 2, 0, 3)
    ).astype(jnp.int32)

    ones_cols = jnp.ones((N_NODES, 16), jnp.float32)
    x_aug = jnp.concatenate([x, ones_cols], axis=1)
    z144 = jnp.zeros((N_NODES, D_FEAT + 16), jnp.float32)
    p1 = _sc_agg_144(x_aug, ei, z144)
    h1 = _tc_layer1(x, p1, W1)

    dp = p1[:, :, D_FEAT:D_FEAT + 1]
    z128 = jnp.zeros((N_NODES, D_FEAT), jnp.float32)
    p2 = _sc_agg_128(h1, ei, z128)
    h2 = _tc_layer2(h1, p2, dp, W2)
    return h2


# speedup vs baseline: 7.1148x; 7.1148x over previous
import functools

import jax
import jax.numpy as jnp
from jax import lax
from jax.experimental import pallas as pl
from jax.experimental.pallas import tpu as pltpu
from jax.experimental.pallas import tpu_sc as plsc

N_NODES = 10000
N_EDGES = 320000
D_FEAT = 128
OUT_SIZE = 128

NC = 2
NS = 16
NW = NC * NS
EPW = N_EDGES // NW
K = 80
NCHUNK = EPW // K
RPT = N_NODES // NS


def _make_sc_agg(dt):
    mesh = plsc.VectorSubcoreMesh(core_axis_name="c", subcore_axis_name="s")

    @functools.partial(
        pl.kernel,
        mesh=mesh,
        out_type=jax.ShapeDtypeStruct((NC, N_NODES, dt), jnp.float32),
        scratch_types=[
            pltpu.VMEM((NCHUNK, 2, K), jnp.int32),
            pltpu.VMEM((K, dt), jnp.float32),
            pltpu.VMEM_SHARED((N_NODES, dt), jnp.float32),
            pltpu.SemaphoreType.DMA,
        ],
        compiler_params=pltpu.CompilerParams(use_tc_tiling_on_sc=False),
    )
    def sc_agg(tab_h, ei_h, z_h, out_p, idx_v, rows_v, acc_sh, sem):
        c = lax.axis_index("c")
        s = lax.axis_index("s")
        wid = c * NS + s
        rows = pl.ds(s * RPT, RPT)

        pltpu.sync_copy(z_h.at[rows], acc_sh.at[rows])
        pltpu.sync_copy(ei_h.at[wid], idx_v)
        plsc.subcore_barrier()

        def chunk(i, carry):
            src_idx = idx_v.at[i, 0]
            dst_idx = idx_v.at[i, 1]
            pltpu.async_copy(tab_h.at[src_idx], rows_v, sem).wait()
            pltpu.sync_copy(rows_v, acc_sh.at[dst_idx], add=True)
            return carry

        lax.fori_loop(0, NCHUNK, chunk, 0)

        plsc.subcore_barrier()
        pltpu.sync_copy(acc_sh.at[rows], out_p.at[c, rows])

    return sc_agg


_sc_agg_144 = _make_sc_agg(D_FEAT + 16)
_sc_agg_128 = _make_sc_agg(D_FEAT)

_TC_R = 2000


def _tc1_body(h_ref, p_ref, w_ref, o_ref):
    h = h_ref[...]
    p = p_ref[0] + p_ref[1]
    d = jnp.maximum(p[:, D_FEAT:D_FEAT + 1], 1e-12)
    agg = p[:, :D_FEAT] / d
    wh = w_ref[:, :D_FEAT]
    wa = w_ref[:, D_FEAT:]
    acc = lax.dot_general(h, wh, (((1,), (1,)), ((), ())),
                          preferred_element_type=jnp.float32)
    acc = acc + lax.dot_general(agg, wa, (((1,), (1,)), ((), ())),
                                preferred_element_type=jnp.float32)
    o_ref[...] = jnp.maximum(acc, 0.0)


def _tc2_body(h_ref, p_ref, dp_ref, w_ref, o_ref):
    h = h_ref[...]
    p = p_ref[0] + p_ref[1]
    d = jnp.maximum(dp_ref[0] + dp_ref[1], 1e-12)
    agg = p / d
    wh = w_ref[:, :D_FEAT]
    wa = w_ref[:, D_FEAT:]
    acc = lax.dot_general(h, wh, (((1,), (1,)), ((), ())),
                          preferred_element_type=jnp.float32)
    acc = acc + lax.dot_general(agg, wa, (((1,), (1,)), ((), ())),
                                preferred_element_type=jnp.float32)
    o_ref[...] = jnp.maximum(acc, 0.0)


def _tc_layer1(h, p, W):
    grid = (N_NODES // _TC_R,)
    return pl.pallas_call(
        _tc1_body,
        grid=grid,
        in_specs=[
            pl.BlockSpec((_TC_R, D_FEAT), lambda i: (i, 0)),
            pl.BlockSpec((NC, _TC_R, D_FEAT + 16), lambda i: (0, i, 0)),
            pl.BlockSpec((OUT_SIZE, 2 * D_FEAT), lambda i: (0, 0)),
        ],
        out_specs=pl.BlockSpec((_TC_R, OUT_SIZE), lambda i: (i, 0)),
        out_shape=jax.ShapeDtypeStruct((N_NODES, OUT_SIZE), jnp.float32),
    )(h, p, W)


def _tc_layer2(h, p, dp, W):
    grid = (N_NODES // _TC_R,)
    return pl.pallas_call(
        _tc2_body,
        grid=grid,
        in_specs=[
            pl.BlockSpec((_TC_R, D_FEAT), lambda i: (i, 0)),
            pl.BlockSpec((NC, _TC_R, D_FEAT), lambda i: (0, i, 0)),
            pl.BlockSpec((NC, _TC_R, 1), lambda i: (0, i, 0)),
            pl.BlockSpec((OUT_SIZE, 2 * OUT_SIZE), lambda i: (0, 0)),
        ],
        out_specs=pl.BlockSpec((_TC_R, OUT_SIZE), lambda i: (i, 0)),
        out_shape=jax.ShapeDtypeStruct((N_NODES, OUT_SIZE), jnp.float32),
    )(h, p, dp, W)


def kernel(x, W1, W2, edge_index):
    ei = jnp.transpose(
        edge_index.reshape(2, NW, NCHUNK, K), (1, 2, 0, 3)
    ).astype(jnp.int32)

    ones_cols = jnp.ones((N_NODES, 16), jnp.float32)
    x_aug = jnp.concatenate([x, ones_cols], axis=1)
    z144 = jnp.zeros((N_NODES, D_FEAT + 16), jnp.float32)
    p1 = _sc_agg_144(x_aug, ei, z144)
    h1 = _tc_layer1(x, p1, W1)

    dp = p1[:, :, D_FEAT:D_FEAT + 1]
    z128 = jnp.zeros((N_NODES, D_FEAT), jnp.float32)
    p2 = _sc_agg_128(h1, ei, z128)
    h2 = _tc_layer2(h1, p2, dp, W2)
    return h2


# speedup vs baseline: 9.1873x; 1.2913x over previous
import functools

import jax
import jax.numpy as jnp
from jax import lax
from jax.experimental import pallas as pl
from jax.experimental.pallas import tpu as pltpu
from jax.experimental.pallas import tpu_sc as plsc

N_NODES = 10000
N_EDGES = 320000
D_FEAT = 128
OUT_SIZE = 128

NC = 2
NS = 16
NW = NC * NS
EPW = N_EDGES // NW
RPT = N_NODES // NS


def _make_sc_agg(dt, k):
    nchunk = EPW // k
    mesh = plsc.VectorSubcoreMesh(core_axis_name="c", subcore_axis_name="s")

    @functools.partial(
        pl.kernel,
        mesh=mesh,
        out_type=jax.ShapeDtypeStruct((NC, N_NODES, dt), jnp.float32),
        scratch_types=[
            pltpu.VMEM((nchunk, 2, k), jnp.int32),
            pltpu.VMEM((2, k, dt), jnp.float32),
            pltpu.VMEM_SHARED((N_NODES, dt), jnp.float32),
            pltpu.SemaphoreType.DMA((2,)),
        ],
        compiler_params=pltpu.CompilerParams(use_tc_tiling_on_sc=False),
    )
    def sc_agg(tab_h, ei_h, z_h, out_p, idx_v, rows_v, acc_sh, sem):
        c = lax.axis_index("c")
        s = lax.axis_index("s")
        wid = c * NS + s
        rows = pl.ds(s * RPT, RPT)

        pltpu.sync_copy(z_h.at[rows], acc_sh.at[rows])
        pltpu.sync_copy(ei_h.at[wid], idx_v)

        def gather(j, b):
            pltpu.async_copy(tab_h.at[idx_v.at[j, 0]], rows_v.at[b], sem.at[b])

        gather(0, 0)
        plsc.subcore_barrier()

        def chunk(i, carry):
            b = lax.rem(i, 2)
            nxt = i + 1

            @pl.when(nxt < nchunk)
            def _():
                gather(nxt, 1 - b)

            pltpu.make_async_copy(
                tab_h.at[idx_v.at[i, 0]], rows_v.at[b], sem.at[b]
            ).wait()
            pltpu.sync_copy(rows_v.at[b], acc_sh.at[idx_v.at[i, 1]], add=True)
            return carry

        lax.fori_loop(0, nchunk, chunk, 0)

        plsc.subcore_barrier()
        pltpu.sync_copy(acc_sh.at[rows], out_p.at[c, rows])

    return sc_agg


K1 = 40
K2 = 80
_sc_agg_144 = _make_sc_agg(D_FEAT + 16, K1)
_sc_agg_128 = _make_sc_agg(D_FEAT, K2)

_TC_R = 2000


def _tc1_body(h_ref, p_ref, w_ref, o_ref):
    h = h_ref[...]
    p = p_ref[0] + p_ref[1]
    d = jnp.maximum(p[:, D_FEAT:D_FEAT + 1], 1e-12)
    agg = p[:, :D_FEAT] / d
    wh = w_ref[:, :D_FEAT]
    wa = w_ref[:, D_FEAT:]
    acc = lax.dot_general(h, wh, (((1,), (1,)), ((), ())),
                          preferred_element_type=jnp.float32)
    acc = acc + lax.dot_general(agg, wa, (((1,), (1,)), ((), ())),
                                preferred_element_type=jnp.float32)
    o_ref[...] = jnp.maximum(acc, 0.0)


def _tc2_body(h_ref, p_ref, dp_ref, w_ref, o_ref):
    h = h_ref[...]
    p = p_ref[0] + p_ref[1]
    d = jnp.maximum(dp_ref[0] + dp_ref[1], 1e-12)
    agg = p / d
    wh = w_ref[:, :D_FEAT]
    wa = w_ref[:, D_FEAT:]
    acc = lax.dot_general(h, wh, (((1,), (1,)), ((), ())),
                          preferred_element_type=jnp.float32)
    acc = acc + lax.dot_general(agg, wa, (((1,), (1,)), ((), ())),
                                preferred_element_type=jnp.float32)
    o_ref[...] = jnp.maximum(acc, 0.0)


def _tc_layer1(h, p, W):
    grid = (N_NODES // _TC_R,)
    return pl.pallas_call(
        _tc1_body,
        grid=grid,
        in_specs=[
            pl.BlockSpec((_TC_R, D_FEAT), lambda i: (i, 0)),
            pl.BlockSpec((NC, _TC_R, D_FEAT + 16), lambda i: (0, i, 0)),
            pl.BlockSpec((OUT_SIZE, 2 * D_FEAT), lambda i: (0, 0)),
        ],
        out_specs=pl.BlockSpec((_TC_R, OUT_SIZE), lambda i: (i, 0)),
        out_shape=jax.ShapeDtypeStruct((N_NODES, OUT_SIZE), jnp.float32),
    )(h, p, W)


def _tc_layer2(h, p, dp, W):
    grid = (N_NODES // _TC_R,)
    return pl.pallas_call(
        _tc2_body,
        grid=grid,
        in_specs=[
            pl.BlockSpec((_TC_R, D_FEAT), lambda i: (i, 0)),
            pl.BlockSpec((NC, _TC_R, D_FEAT), lambda i: (0, i, 0)),
            pl.BlockSpec((NC, _TC_R, 1), lambda i: (0, i, 0)),
            pl.BlockSpec((OUT_SIZE, 2 * OUT_SIZE), lambda i: (0, 0)),
        ],
        out_specs=pl.BlockSpec((_TC_R, OUT_SIZE), lambda i: (i, 0)),
        out_shape=jax.ShapeDtypeStruct((N_NODES, OUT_SIZE), jnp.float32),
    )(h, p, dp, W)


def kernel(x, W1, W2, edge_index):
    ei32 = edge_index.astype(jnp.int32)
    ei1 = jnp.transpose(ei32.reshape(2, NW, EPW // K1, K1), (1, 2, 0, 3))
    ei2 = jnp.transpose(ei32.reshape(2, NW, EPW // K2, K2), (1, 2, 0, 3))

    ones_cols = jnp.ones((N_NODES, 16), jnp.float32)
    x_aug = jnp.concatenate([x, ones_cols], axis=1)
    z144 = jnp.zeros((N_NODES, D_FEAT + 16), jnp.float32)
    p1 = _sc_agg_144(x_aug, ei1, z144)
    h1 = _tc_layer1(x, p1, W1)

    dp = p1[:, :, D_FEAT:D_FEAT + 1]
    z128 = jnp.zeros((N_NODES, D_FEAT), jnp.float32)
    p2 = _sc_agg_128(h1, ei2, z128)
    h2 = _tc_layer2(h1, p2, dp, W2)
    return h2
